# baseline (speedup 1.0000x reference)
import jax
import jax.numpy as jnp
from jax.experimental import pallas as pl
from jax.experimental.pallas import tpu as pltpu


def _l1_body(g_ref, f_ref, w_ref, h1_ref, q_ref, s_ref):
    @pl.when(pl.program_id(0) == 0)
    def _():
        s_ref[...] = jnp.dot(f_ref[...], w_ref[...],
                             preferred_element_type=jnp.float32)

    gblk = g_ref[...]
    m = jnp.dot(gblk, s_ref[...], preferred_element_type=jnp.float32)
    h1_ref[...] = jnp.tanh(m)
    q_ref[...] = jnp.round(gblk * 255.0).astype(jnp.uint8)


def _mid_body(q_ref, x_ref, w_ref, out_ref, shi_ref, slo_ref):
    @pl.when(pl.program_id(0) == 0)
    def _():
        s = jnp.dot(x_ref[...], w_ref[...],
                    preferred_element_type=jnp.float32) * (1.0 / 255.0)
        hi = s.astype(jnp.bfloat16)
        shi_ref[...] = hi
        slo_ref[...] = (s - hi.astype(jnp.float32)).astype(jnp.bfloat16)

    qb = q_ref[...].astype(jnp.bfloat16)
    acc = jnp.dot(qb, shi_ref[...], preferred_element_type=jnp.float32)
    acc = acc + jnp.dot(qb, slo_ref[...], preferred_element_type=jnp.float32)
    out_ref[...] = acc


def _dec_body(a_ref, b_ref, adj_ref):
    adj_ref[...] = jax.lax.dot_general(
        a_ref[...], b_ref[...], (((1,), (1,)), ((), ())),
        preferred_element_type=jnp.float32)


def _mid_layer(q, x, w, block_m):
    n = q.shape[0]
    k = x.shape[1]
    ko = w.shape[1]
    return pl.pallas_call(
        _mid_body,
        grid=(n // block_m,),
        in_specs=[
            pl.BlockSpec((block_m, n), lambda i: (i, 0)),
            pl.BlockSpec((n, k), lambda i: (0, 0)),
            pl.BlockSpec((k, ko), lambda i: (0, 0)),
        ],
        out_specs=pl.BlockSpec((block_m, ko), lambda i: (i, 0)),
        out_shape=jax.ShapeDtypeStruct((n, ko), jnp.float32),
        scratch_shapes=[
            pltpu.VMEM((n, ko), jnp.bfloat16),
            pltpu.VMEM((n, ko), jnp.bfloat16),
        ],
    )(q, x, w)


def kernel(g, f, W1, W2, W3):
    n = g.shape[0]
    d0 = f.shape[1]
    d1 = W1.shape[1]

    block_m = 400
    h1, q = pl.pallas_call(
        _l1_body,
        grid=(n // block_m,),
        in_specs=[
            pl.BlockSpec((block_m, n), lambda i: (i, 0)),
            pl.BlockSpec((n, d0), lambda i: (0, 0)),
            pl.BlockSpec((d0, d1), lambda i: (0, 0)),
        ],
        out_specs=[
            pl.BlockSpec((block_m, d1), lambda i: (i, 0)),
            pl.BlockSpec((block_m, n), lambda i: (i, 0)),
        ],
        out_shape=[
            jax.ShapeDtypeStruct((n, d1), jnp.float32),
            jax.ShapeDtypeStruct((n, n), jnp.uint8),
        ],
        scratch_shapes=[pltpu.VMEM((n, d1), jnp.float32)],
    )(g, f, W1)

    h2 = _mid_layer(q, h1, W2, block_m)
    h3 = _mid_layer(q, h2, W3, block_m)

    block_r, block_c = 2000, 2048
    adj = pl.pallas_call(
        _dec_body,
        grid=(n // block_r, pl.cdiv(n, block_c)),
        in_specs=[
            pl.BlockSpec((block_r, W3.shape[1]), lambda i, j: (i, 0)),
            pl.BlockSpec((block_c, W3.shape[1]), lambda i, j: (j, 0)),
        ],
        out_specs=pl.BlockSpec((block_r, block_c), lambda i, j: (i, j)),
        out_shape=jax.ShapeDtypeStruct((n, n), jnp.float32),
    )(h3, h3)

    return (h1, h3, adj, h2, h3)


# speedup vs baseline: 1.1272x; 1.1272x over previous
import jax
import jax.numpy as jnp
from jax.experimental import pallas as pl
from jax.experimental.pallas import tpu as pltpu


def _l1_body(g_ref, f_ref, w_ref, h1_ref, gb_ref, s_ref):
    @pl.when(pl.program_id(0) == 0)
    def _():
        s_ref[...] = jnp.dot(f_ref[...], w_ref[...],
                             preferred_element_type=jnp.float32)

    gblk = g_ref[...]
    m = jnp.dot(gblk, s_ref[...], preferred_element_type=jnp.float32)
    h1_ref[...] = jnp.tanh(m)
    gb_ref[...] = gblk.astype(jnp.bfloat16)


def _mid_body(gb_ref, x_ref, w_ref, out_ref, s_ref):
    @pl.when(pl.program_id(0) == 0)
    def _():
        s = jnp.dot(x_ref[...], w_ref[...],
                    preferred_element_type=jnp.float32)
        hi = s.astype(jnp.bfloat16)
        lo = (s - hi.astype(jnp.float32)).astype(jnp.bfloat16)
        s_ref[...] = jnp.concatenate([hi, lo], axis=1)

    acc = jnp.dot(gb_ref[...], s_ref[...], preferred_element_type=jnp.float32)
    ko = out_ref.shape[1]
    out_ref[...] = acc[:, :ko] + acc[:, ko:]


def _dec_body(a_ref, b_ref, adj_ref):
    adj_ref[...] = jax.lax.dot_general(
        a_ref[...], b_ref[...], (((1,), (1,)), ((), ())),
        preferred_element_type=jnp.float32)


def _mid_layer(gb, x, w, block_m):
    n = gb.shape[0]
    k = x.shape[1]
    ko = w.shape[1]
    return pl.pallas_call(
        _mid_body,
        grid=(n // block_m,),
        in_specs=[
            pl.BlockSpec((block_m, n), lambda i: (i, 0)),
            pl.BlockSpec((n, k), lambda i: (0, 0)),
            pl.BlockSpec((k, ko), lambda i: (0, 0)),
        ],
        out_specs=pl.BlockSpec((block_m, ko), lambda i: (i, 0)),
        out_shape=jax.ShapeDtypeStruct((n, ko), jnp.float32),
        scratch_shapes=[pltpu.VMEM((n, 2 * ko), jnp.bfloat16)],
    )(gb, x, w)


def kernel(g, f, W1, W2, W3):
    n = g.shape[0]
    d0 = f.shape[1]
    d1 = W1.shape[1]

    block_m = 400
    h1, gb = pl.pallas_call(
        _l1_body,
        grid=(n // block_m,),
        in_specs=[
            pl.BlockSpec((block_m, n), lambda i: (i, 0)),
            pl.BlockSpec((n, d0), lambda i: (0, 0)),
            pl.BlockSpec((d0, d1), lambda i: (0, 0)),
        ],
        out_specs=[
            pl.BlockSpec((block_m, d1), lambda i: (i, 0)),
            pl.BlockSpec((block_m, n), lambda i: (i, 0)),
        ],
        out_shape=[
            jax.ShapeDtypeStruct((n, d1), jnp.float32),
            jax.ShapeDtypeStruct((n, n), jnp.bfloat16),
        ],
        scratch_shapes=[pltpu.VMEM((n, d1), jnp.float32)],
    )(g, f, W1)

    h2 = _mid_layer(gb, h1, W2, block_m)
    h3 = _mid_layer(gb, h2, W3, block_m)

    block_r, block_c = 2000, 2048
    adj = pl.pallas_call(
        _dec_body,
        grid=(n // block_r, pl.cdiv(n, block_c)),
        in_specs=[
            pl.BlockSpec((block_r, W3.shape[1]), lambda i, j: (i, 0)),
            pl.BlockSpec((block_c, W3.shape[1]), lambda i, j: (j, 0)),
        ],
        out_specs=pl.BlockSpec((block_r, block_c), lambda i, j: (i, j)),
        out_shape=jax.ShapeDtypeStruct((n, n), jnp.float32),
    )(h3, h3)

    return (h1, h3, adj, h2, h3)
